# baseline (speedup 1.0000x reference)
import functools

import jax
import jax.numpy as jnp
from jax import lax
from jax.experimental import pallas as pl
from jax.experimental.pallas import tpu as pltpu
from jax.experimental.pallas import tpu_sc as plsc

_N = 8192
_B = 16384


_R = 512


def _lse_block(w_ref, out_ref):
    x = w_ref[...]
    m = jnp.max(x, axis=1, keepdims=True)
    s = jnp.sum(jnp.exp(x - m), axis=1)
    out_ref[...] = jnp.log(s) + m[:, 0]


def _row_lse(w):
    return pl.pallas_call(
        _lse_block,
        grid=(_N // _R,),
        in_specs=[pl.BlockSpec((_R, _N), lambda i: (i, 0))],
        out_specs=pl.BlockSpec((_R,), lambda i: (i,)),
        out_shape=jax.ShapeDtypeStruct((_N,), jnp.float32),
    )(w)


_NC, _NS, _L = 2, 16, 16
_NW = _NC * _NS
_BPW = _B // _NW
_CH = 128
_NCH = _BPW // _CH
_G = 128


def _sc_body(w_hbm, lse, conds3, inp3, out3, c_v, i_v, idx_v, blk_v, lseg_v, o_v, sem):
    wid = lax.axis_index("s") * _NC + lax.axis_index("c")
    w16 = w_hbm
    pltpu.sync_copy(conds3.at[wid], c_v)
    pltpu.sync_copy(inp3.at[wid], i_v)
    for j in range(_NCH):
        for t in range(_CH // _L):
            sl = pl.ds(t * _L, _L)
            idx_v[j, sl] = c_v[j, sl] * (_N // _G) + (i_v[j, sl] >> 7)
    for j in range(_NCH):
        pltpu.async_copy(w16.at[idx_v.at[j]], blk_v.at[pl.ds(j * _CH, _CH)], sem).wait()
        pltpu.async_copy(lse.at[c_v.at[j]], lseg_v.at[j], sem).wait()
    lane = jax.lax.iota(jnp.int32, _L)
    for j in range(_NCH):
        for t in range(_CH // _L):
            sl = pl.ds(t * _L, _L)
            v = plsc.load_gather(
                blk_v, [lane + (j * _CH + t * _L), i_v[j, sl] & (_G - 1)])
            o_v[j, sl] = v - lseg_v[j, sl]
    pltpu.sync_copy(o_v, out3.at[wid])


def _sc_gather(w_2d, lse, conds3, inp3):
    mesh = plsc.VectorSubcoreMesh(core_axis_name="c", subcore_axis_name="s")
    return pl.kernel(
        _sc_body,
        out_type=jax.ShapeDtypeStruct((_NW, _NCH, _CH), jnp.float32),
        mesh=mesh,
        compiler_params=pltpu.CompilerParams(needs_layout_passes=False),
        scratch_types=[
            pltpu.VMEM((_NCH, _CH), jnp.int32),
            pltpu.VMEM((_NCH, _CH), jnp.int32),
            pltpu.VMEM((_NCH, _CH), jnp.int32),
            pltpu.VMEM((_BPW, _G), jnp.float32),
            pltpu.VMEM((_NCH, _CH), jnp.float32),
            pltpu.VMEM((_NCH, _CH), jnp.float32),
            pltpu.SemaphoreType.DMA,
        ],
    )(w_2d, lse, conds3, inp3)


def kernel(inputs, conds, w):
    conds_ = conds.reshape(_NW, _NCH, _CH).astype(jnp.int32)
    inp_ = inputs.reshape(_NW, _NCH, _CH).astype(jnp.int32)
    lse = _row_lse(w)
    out3 = _sc_gather(w.reshape(_N * _N // _G, _G), lse, conds_, inp_)
    return out3.reshape(_B)


# speedup vs baseline: 3.1387x; 3.1387x over previous
import functools

import jax
import jax.numpy as jnp
from jax import lax
from jax.experimental import pallas as pl
from jax.experimental.pallas import tpu as pltpu
from jax.experimental.pallas import tpu_sc as plsc

_N = 8192
_B = 16384


_R = 512


def _lse_block(w_ref, out_ref):
    x = w_ref[...]
    m = jnp.max(x, axis=1, keepdims=True)
    s = jnp.sum(jnp.exp(x - m), axis=1)
    out_ref[...] = jnp.log(s) + m[:, 0]


def _row_lse(w):
    return pl.pallas_call(
        _lse_block,
        grid=(_N // _R,),
        in_specs=[pl.BlockSpec((_R, _N), lambda i: (i, 0))],
        out_specs=pl.BlockSpec((_R,), lambda i: (i,)),
        out_shape=jax.ShapeDtypeStruct((_N,), jnp.float32),
    )(w)


_NC, _NS, _L = 2, 16, 16
_NW = _NC * _NS
_BPW = _B // _NW
_GRP = 64
_NGRP = _BPW // _GRP


def _vals_body(w_hbm, conds2, inp2, vals2, c_v, i_v, blk_v, o_v, sem):
    wid = lax.axis_index("s") * _NC + lax.axis_index("c")
    pltpu.sync_copy(conds2.at[wid], c_v)
    pltpu.sync_copy(inp2.at[wid], i_v)
    lane = lax.iota(jnp.int32, _L)

    def group(g, _):
        base = g * _GRP
        copies = []
        for kk in range(_GRP // _L):
            cv = c_v[pl.ds(base + kk * _L, _L)]
            iv = i_v[pl.ds(base + kk * _L, _L)]
            r8 = (cv >> 3) << 3
            cb = (iv >> 7) << 7
            for l in range(_L):
                ro = pl.multiple_of(r8[l], 8)
                co = pl.multiple_of(cb[l], 128)
                copies.append(pltpu.async_copy(
                    w_hbm.at[pl.ds(ro, 8), pl.ds(co, 128)],
                    blk_v.at[kk * _L + l], sem))
        for cp in copies:
            cp.wait()
        for t in range(_GRP // _L):
            sl = pl.ds(base + t * _L, _L)
            v = plsc.load_gather(
                blk_v, [lane + t * _L, c_v[sl] & 7, i_v[sl] & 127])
            o_v[sl] = v
        return 0

    lax.fori_loop(0, _NGRP, group, 0, unroll=False)
    pltpu.sync_copy(o_v, vals2.at[wid])


def _comb_body(lse, conds2, vals2, out2, c_v, v_v, lseg_v, o_v, sem):
    wid = lax.axis_index("s") * _NC + lax.axis_index("c")
    pltpu.sync_copy(conds2.at[wid], c_v)
    pltpu.sync_copy(vals2.at[wid], v_v)
    for j in range(_BPW // 128):
        pltpu.async_copy(
            lse.at[c_v.at[pl.ds(j * 128, 128)]],
            lseg_v.at[pl.ds(j * 128, 128)], sem).wait()
    for t in range(_BPW // _L):
        sl = pl.ds(t * _L, _L)
        o_v[sl] = v_v[sl] - lseg_v[sl]
    pltpu.sync_copy(o_v, out2.at[wid])


def _sc_mesh():
    return plsc.VectorSubcoreMesh(core_axis_name="c", subcore_axis_name="s")


def _sc_vals(w, conds2, inp2):
    return pl.kernel(
        _vals_body,
        out_type=jax.ShapeDtypeStruct((_NW, _BPW), jnp.float32),
        mesh=_sc_mesh(),
        compiler_params=pltpu.CompilerParams(needs_layout_passes=False),
        scratch_types=[
            pltpu.VMEM((_BPW,), jnp.int32),
            pltpu.VMEM((_BPW,), jnp.int32),
            pltpu.VMEM((_GRP, 8, 128), jnp.float32),
            pltpu.VMEM((_BPW,), jnp.float32),
            pltpu.SemaphoreType.DMA,
        ],
    )(w, conds2, inp2)


def _sc_combine(lse, conds2, vals2):
    return pl.kernel(
        _comb_body,
        out_type=jax.ShapeDtypeStruct((_NW, _BPW), jnp.float32),
        mesh=_sc_mesh(),
        compiler_params=pltpu.CompilerParams(needs_layout_passes=False),
        scratch_types=[
            pltpu.VMEM((_BPW,), jnp.int32),
            pltpu.VMEM((_BPW,), jnp.float32),
            pltpu.VMEM((_BPW,), jnp.float32),
            pltpu.VMEM((_BPW,), jnp.float32),
            pltpu.SemaphoreType.DMA,
        ],
    )(lse, conds2, vals2)


def kernel(inputs, conds, w):
    conds2 = conds.reshape(_NW, _BPW).astype(jnp.int32)
    inp2 = inputs.reshape(_NW, _BPW).astype(jnp.int32)
    vals2 = _sc_vals(w, conds2, inp2)
    lse = _row_lse(w)
    out2 = _sc_combine(lse, conds2, vals2)
    return out2.reshape(_B)
